# baseline (speedup 1.0000x reference)
import functools

import jax
import jax.numpy as jnp
from jax import lax
from jax.experimental import pallas as pl
from jax.experimental.pallas import tpu as pltpu
from jax.experimental.pallas import tpu_sc as plsc

DIM = 32
BATCH = 16384
NUM_CORES = 2
NUM_SUBCORES = 16
LANES = 16
NUM_WORKERS = NUM_CORES * NUM_SUBCORES
BW = BATCH // NUM_WORKERS
IDX_CHUNK = 128
NCHUNK = BW // IDX_CHUNK
GROUPS = BW // LANES


def _mf_body(uid_hbm, iid_hbm, uemb_hbm, iemb_hbm, out_hbm,
             uidx_v, iidx_v, urows_v, irows_v, out_v, sem):
    wid = lax.axis_index("s") * NUM_CORES + lax.axis_index("c")
    base = wid * BW

    for j in range(NCHUNK):
        pltpu.sync_copy(uid_hbm.at[pl.ds(base + j * IDX_CHUNK, IDX_CHUNK)],
                        uidx_v.at[j])
        pltpu.sync_copy(iid_hbm.at[pl.ds(base + j * IDX_CHUNK, IDX_CHUNK)],
                        iidx_v.at[j])

    copies = []
    for j in range(NCHUNK):
        copies.append(pltpu.async_copy(
            uemb_hbm.at[uidx_v.at[j]],
            urows_v.at[pl.ds(j * IDX_CHUNK, IDX_CHUNK), :], sem))
        copies.append(pltpu.async_copy(
            iemb_hbm.at[iidx_v.at[j]],
            irows_v.at[pl.ds(j * IDX_CHUNK, IDX_CHUNK), :], sem))
    for c in copies:
        c.wait()

    lane = lax.iota(jnp.int32, LANES)

    def group_body(g, carry):
        row = g * LANES + lane
        acc = jnp.zeros((LANES,), jnp.float32)
        for d in range(DIM):
            col = jnp.full((LANES,), d, jnp.int32)
            u_d = plsc.load_gather(urows_v, [row, col])
            i_d = plsc.load_gather(irows_v, [row, col])
            acc = acc + u_d * i_d
        out_v[pl.ds(g * LANES, LANES)] = acc
        return carry

    lax.fori_loop(0, GROUPS, group_body, 0)

    pltpu.sync_copy(out_v, out_hbm.at[pl.ds(base, BW)])


@jax.jit
def _mf_sc(uid_batch, iid_batch, user_embedding, item_embedding):
    mesh = plsc.VectorSubcoreMesh(core_axis_name="c", subcore_axis_name="s")
    run = functools.partial(
        pl.kernel,
        out_type=jax.ShapeDtypeStruct((BATCH,), jnp.float32),
        mesh=mesh,
        compiler_params=pltpu.CompilerParams(needs_layout_passes=False,
                                             use_tc_tiling_on_sc=False),
        scratch_types=[
            pltpu.VMEM((NCHUNK, IDX_CHUNK), jnp.int32),
            pltpu.VMEM((NCHUNK, IDX_CHUNK), jnp.int32),
            pltpu.VMEM((BW, DIM), jnp.float32),
            pltpu.VMEM((BW, DIM), jnp.float32),
            pltpu.VMEM((BW,), jnp.float32),
            pltpu.SemaphoreType.DMA,
        ],
    )(_mf_body)
    return run(uid_batch, iid_batch, user_embedding, item_embedding)


def kernel(uid_batch, iid_batch, user_embedding, item_embedding):
    return _mf_sc(uid_batch.astype(jnp.int32), iid_batch.astype(jnp.int32),
                  user_embedding, item_embedding)


# speedup vs baseline: 3.4250x; 3.4250x over previous
import functools

import jax
import jax.numpy as jnp
from jax import lax
from jax.experimental import pallas as pl
from jax.experimental.pallas import tpu as pltpu
from jax.experimental.pallas import tpu_sc as plsc

DIM = 32
BATCH = 16384
NUM_CORES = 2
NUM_SUBCORES = 16
LANES = 16
NUM_WORKERS = NUM_CORES * NUM_SUBCORES
BW = BATCH // NUM_WORKERS
CH = 4
NCHUNK = BW // CH
GROUPS = BW // LANES


def _fire_chunk(t, uidx_v, iidx_v, uembt_hbm, iembt_hbm, ublk, iblk, sem):
    uvec = uidx_v[pl.ds(t * CH, LANES)]
    ivec = iidx_v[pl.ds(t * CH, LANES)]
    for l in range(CH):
        u = uvec[l]
        it = ivec[l]
        u128 = pl.multiple_of((u >> 7) << 7, 128)
        i128 = pl.multiple_of((it >> 7) << 7, 128)
        pltpu.async_copy(uembt_hbm.at[pl.ds(0, DIM), pl.ds(u128, 128)],
                         ublk.at[:, pl.ds(l * 128, 128)], sem)
        pltpu.async_copy(iembt_hbm.at[pl.ds(0, DIM), pl.ds(i128, 128)],
                         iblk.at[:, pl.ds(l * 128, 128)], sem)


def _drain_chunk(uembt_hbm, ublk, iblk, sem):
    dummy = uembt_hbm.at[pl.ds(0, DIM), pl.ds(0, CH * 128)]
    pltpu.make_async_copy(dummy, ublk, sem).wait()
    pltpu.make_async_copy(dummy, iblk, sem).wait()


def _extract_chunk(t, uidx_v, iidx_v, ublk, iblk, ucomp, icomp):
    cvec = lax.iota(jnp.int32, LANES)
    uvec = uidx_v[pl.ds(t * CH, LANES)]
    ivec = iidx_v[pl.ds(t * CH, LANES)]
    for l in range(CH):
        ucol = jnp.full((LANES,), l * 128, jnp.int32) + (uvec[l] & 127)
        icol = jnp.full((LANES,), l * 128, jnp.int32) + (ivec[l] & 127)
        dst = (t * CH + l) * DIM
        ucomp[pl.ds(dst, LANES)] = plsc.load_gather(ublk, [cvec, ucol])
        ucomp[pl.ds(dst + LANES, LANES)] = plsc.load_gather(
            ublk, [cvec + LANES, ucol])
        icomp[pl.ds(dst, LANES)] = plsc.load_gather(iblk, [cvec, icol])
        icomp[pl.ds(dst + LANES, LANES)] = plsc.load_gather(
            iblk, [cvec + LANES, icol])


def _mf_body(uid_hbm, iid_hbm, uembt_hbm, iembt_hbm, out_hbm,
             uidx_v, iidx_v, ublk_a, iblk_a, ublk_b, iblk_b,
             ucomp, icomp, out_v, sem_a, sem_b):
    wid = lax.axis_index("s") * NUM_CORES + lax.axis_index("c")
    base = wid * BW

    pltpu.sync_copy(uid_hbm.at[pl.ds(base, BW)], uidx_v.at[pl.ds(0, BW)])
    pltpu.sync_copy(iid_hbm.at[pl.ds(base, BW)], iidx_v.at[pl.ds(0, BW)])

    _fire_chunk(0, uidx_v, iidx_v, uembt_hbm, iembt_hbm, ublk_a, iblk_a,
                sem_a)
    _fire_chunk(1, uidx_v, iidx_v, uembt_hbm, iembt_hbm, ublk_b, iblk_b,
                sem_b)

    def loop_body(j, carry):
        _drain_chunk(uembt_hbm, ublk_a, iblk_a, sem_a)
        _extract_chunk(2 * j, uidx_v, iidx_v, ublk_a, iblk_a, ucomp, icomp)
        _fire_chunk(2 * j + 2, uidx_v, iidx_v, uembt_hbm, iembt_hbm,
                    ublk_a, iblk_a, sem_a)
        _drain_chunk(uembt_hbm, ublk_b, iblk_b, sem_b)
        _extract_chunk(2 * j + 1, uidx_v, iidx_v, ublk_b, iblk_b,
                       ucomp, icomp)
        _fire_chunk(2 * j + 3, uidx_v, iidx_v, uembt_hbm, iembt_hbm,
                    ublk_b, iblk_b, sem_b)
        return carry

    lax.fori_loop(0, NCHUNK // 2 - 1, loop_body, 0)

    _drain_chunk(uembt_hbm, ublk_a, iblk_a, sem_a)
    _extract_chunk(NCHUNK - 2, uidx_v, iidx_v, ublk_a, iblk_a, ucomp, icomp)
    _drain_chunk(uembt_hbm, ublk_b, iblk_b, sem_b)
    _extract_chunk(NCHUNK - 1, uidx_v, iidx_v, ublk_b, iblk_b, ucomp, icomp)

    lane = lax.iota(jnp.int32, LANES)

    def group_body(g, carry):
        flat_base = (g * LANES + lane) * DIM
        acc = jnp.zeros((LANES,), jnp.float32)
        for d in range(DIM):
            u_d = plsc.load_gather(ucomp, [flat_base + d])
            i_d = plsc.load_gather(icomp, [flat_base + d])
            acc = acc + u_d * i_d
        out_v[pl.ds(g * LANES, LANES)] = acc
        return carry

    lax.fori_loop(0, GROUPS, group_body, 0)

    pltpu.sync_copy(out_v, out_hbm.at[pl.ds(base, BW)])


@jax.jit
def _mf_sc(uid_batch, iid_batch, user_embedding, item_embedding):
    mesh = plsc.VectorSubcoreMesh(core_axis_name="c", subcore_axis_name="s")
    run = functools.partial(
        pl.kernel,
        out_type=jax.ShapeDtypeStruct((BATCH,), jnp.float32),
        mesh=mesh,
        compiler_params=pltpu.CompilerParams(needs_layout_passes=False),
        scratch_types=[
            pltpu.VMEM((BW + LANES,), jnp.int32),
            pltpu.VMEM((BW + LANES,), jnp.int32),
            pltpu.VMEM((DIM, CH * 128), jnp.float32),
            pltpu.VMEM((DIM, CH * 128), jnp.float32),
            pltpu.VMEM((DIM, CH * 128), jnp.float32),
            pltpu.VMEM((DIM, CH * 128), jnp.float32),
            pltpu.VMEM((BW * DIM,), jnp.float32),
            pltpu.VMEM((BW * DIM,), jnp.float32),
            pltpu.VMEM((BW,), jnp.float32),
            pltpu.SemaphoreType.DMA,
            pltpu.SemaphoreType.DMA,
        ],
    )(_mf_body)
    return run(uid_batch, iid_batch, user_embedding.T, item_embedding.T)


def kernel(uid_batch, iid_batch, user_embedding, item_embedding):
    return _mf_sc(uid_batch.astype(jnp.int32), iid_batch.astype(jnp.int32),
                  user_embedding, item_embedding)


# speedup vs baseline: 3.6753x; 1.0731x over previous
import functools

import jax
import jax.numpy as jnp
from jax import lax
from jax.experimental import pallas as pl
from jax.experimental.pallas import tpu as pltpu
from jax.experimental.pallas import tpu_sc as plsc

DIM = 32
BATCH = 16384
NUM_CORES = 2
NUM_SUBCORES = 16
LANES = 16
NUM_WORKERS = NUM_CORES * NUM_SUBCORES
BW = BATCH // NUM_WORKERS
CH = 4
NCHUNK = BW // CH
GROUPS = BW // LANES


def _fire_chunk(t, uidx_v, iidx_v, uembt_hbm, iembt_hbm, ublk, iblk, sem):
    uvec = uidx_v[pl.ds(t * CH, LANES)]
    ivec = iidx_v[pl.ds(t * CH, LANES)]
    for l in range(CH):
        u = uvec[l]
        it = ivec[l]
        u128 = pl.multiple_of((u >> 7) << 7, 128)
        i128 = pl.multiple_of((it >> 7) << 7, 128)
        pltpu.async_copy(uembt_hbm.at[pl.ds(0, DIM), pl.ds(u128, 128)],
                         ublk.at[:, pl.ds(l * 128, 128)], sem)
        pltpu.async_copy(iembt_hbm.at[pl.ds(0, DIM), pl.ds(i128, 128)],
                         iblk.at[:, pl.ds(l * 128, 128)], sem)


def _drain_chunk(uembt_hbm, ublk, iblk, sem):
    dummy = uembt_hbm.at[pl.ds(0, DIM), pl.ds(0, CH * 128)]
    pltpu.make_async_copy(dummy, ublk, sem).wait()
    pltpu.make_async_copy(dummy, iblk, sem).wait()


def _extract_chunk(t, uidx_v, iidx_v, ublk, iblk, pcomp):
    cvec = lax.iota(jnp.int32, LANES)
    uvec = uidx_v[pl.ds(t * CH, LANES)]
    ivec = iidx_v[pl.ds(t * CH, LANES)]
    for l in range(CH):
        ucol = jnp.full((LANES,), l * 128, jnp.int32) + (uvec[l] & 127)
        icol = jnp.full((LANES,), l * 128, jnp.int32) + (ivec[l] & 127)
        dst = (t * CH + l) * DIM
        pcomp[pl.ds(dst, LANES)] = (
            plsc.load_gather(ublk, [cvec, ucol])
            * plsc.load_gather(iblk, [cvec, icol]))
        pcomp[pl.ds(dst + LANES, LANES)] = (
            plsc.load_gather(ublk, [cvec + LANES, ucol])
            * plsc.load_gather(iblk, [cvec + LANES, icol]))


def _dot_group(g, pcomp, out_v):
    lane = lax.iota(jnp.int32, LANES)
    flat_base = (g * LANES + lane) * DIM
    acc = jnp.zeros((LANES,), jnp.float32)
    for d in range(DIM):
        acc = acc + plsc.load_gather(pcomp, [flat_base + d])
    out_v[pl.ds(g * LANES, LANES)] = acc


def _mf_body(uid_hbm, iid_hbm, uembt_hbm, iembt_hbm, out_hbm,
             uidx_v, iidx_v, ublk_a, iblk_a, ublk_b, iblk_b,
             pcomp, out_v, sem_a, sem_b):
    wid = lax.axis_index("s") * NUM_CORES + lax.axis_index("c")
    base = wid * BW

    pltpu.sync_copy(uid_hbm.at[pl.ds(base, BW)], uidx_v.at[pl.ds(0, BW)])
    pltpu.sync_copy(iid_hbm.at[pl.ds(base, BW)], iidx_v.at[pl.ds(0, BW)])

    _fire_chunk(0, uidx_v, iidx_v, uembt_hbm, iembt_hbm, ublk_a, iblk_a,
                sem_a)
    _fire_chunk(1, uidx_v, iidx_v, uembt_hbm, iembt_hbm, ublk_b, iblk_b,
                sem_b)

    def loop_body(j, carry):
        _drain_chunk(uembt_hbm, ublk_a, iblk_a, sem_a)
        _extract_chunk(2 * j, uidx_v, iidx_v, ublk_a, iblk_a, pcomp)
        _fire_chunk(2 * j + 2, uidx_v, iidx_v, uembt_hbm, iembt_hbm,
                    ublk_a, iblk_a, sem_a)
        _drain_chunk(uembt_hbm, ublk_b, iblk_b, sem_b)
        _extract_chunk(2 * j + 1, uidx_v, iidx_v, ublk_b, iblk_b, pcomp)
        _fire_chunk(2 * j + 3, uidx_v, iidx_v, uembt_hbm, iembt_hbm,
                    ublk_b, iblk_b, sem_b)

        @pl.when(j % 2 == 1)
        def _():
            _dot_group((j - 1) // 2, pcomp, out_v)

        return carry

    lax.fori_loop(0, NCHUNK // 2 - 1, loop_body, 0)

    _drain_chunk(uembt_hbm, ublk_a, iblk_a, sem_a)
    _extract_chunk(NCHUNK - 2, uidx_v, iidx_v, ublk_a, iblk_a, pcomp)
    _drain_chunk(uembt_hbm, ublk_b, iblk_b, sem_b)
    _extract_chunk(NCHUNK - 1, uidx_v, iidx_v, ublk_b, iblk_b, pcomp)

    def tail_group(g, carry):
        _dot_group(g, pcomp, out_v)
        return carry

    lax.fori_loop(GROUPS - 2, GROUPS, tail_group, 0)

    pltpu.sync_copy(out_v, out_hbm.at[pl.ds(base, BW)])


@jax.jit
def _mf_sc(uid_batch, iid_batch, user_embedding, item_embedding):
    mesh = plsc.VectorSubcoreMesh(core_axis_name="c", subcore_axis_name="s")
    run = functools.partial(
        pl.kernel,
        out_type=jax.ShapeDtypeStruct((BATCH,), jnp.float32),
        mesh=mesh,
        compiler_params=pltpu.CompilerParams(needs_layout_passes=False),
        scratch_types=[
            pltpu.VMEM((BW + LANES,), jnp.int32),
            pltpu.VMEM((BW + LANES,), jnp.int32),
            pltpu.VMEM((DIM, CH * 128), jnp.float32),
            pltpu.VMEM((DIM, CH * 128), jnp.float32),
            pltpu.VMEM((DIM, CH * 128), jnp.float32),
            pltpu.VMEM((DIM, CH * 128), jnp.float32),
            pltpu.VMEM((BW * DIM,), jnp.float32),
            pltpu.VMEM((BW,), jnp.float32),
            pltpu.SemaphoreType.DMA,
            pltpu.SemaphoreType.DMA,
        ],
    )(_mf_body)
    return run(uid_batch, iid_batch, user_embedding.T, item_embedding.T)


def kernel(uid_batch, iid_batch, user_embedding, item_embedding):
    return _mf_sc(uid_batch.astype(jnp.int32), iid_batch.astype(jnp.int32),
                  user_embedding, item_embedding)
